# Initial kernel scaffold; baseline (speedup 1.0000x reference)
import jax
import jax.numpy as jnp
from jax.experimental import pallas as pl


def kernel(x_gene, x_trait, edge_g2t_src, edge_g2t_dst, edge_t2g_src, edge_t2g_dst, in_W, in_b, ln_g, ln_b, Wk, bk, Wq, bq, Wv, bv, Wa, ba, a_rel, m_rel, p_rel, skip, out_W, out_b):
    raise NotImplementedError("write your pallas kernel here")


# speedup vs baseline: 6.4071x; 6.4071x over previous
import jax
import jax.numpy as jnp
from jax.experimental import pallas as pl

_NG, _NT, _E, _HID, _H, _D, _L = 50000, 5000, 300000, 128, 4, 32, 2
_BLK = 1024


def _bs_rows(cols):
    return pl.BlockSpec((_BLK, cols), lambda i: (i, 0))


def _bs_full(r, c):
    return pl.BlockSpec((r, c), lambda i: (0, 0))


def _call(body, n_rows, out_cols, operands, in_specs):
    return pl.pallas_call(
        body,
        grid=(pl.cdiv(n_rows, _BLK),),
        in_specs=in_specs,
        out_specs=_bs_rows(out_cols),
        out_shape=jax.ShapeDtypeStruct((n_rows, out_cols), jnp.float32),
    )(*operands)


def _in_ln_body(x_ref, w_ref, b_ref, g_ref, bb_ref, o_ref):
    y = jnp.dot(x_ref[...], w_ref[...], preferred_element_type=jnp.float32)
    y = y + b_ref[...]
    mu = y.mean(-1, keepdims=True)
    var = ((y - mu) ** 2).mean(-1, keepdims=True)
    o_ref[...] = g_ref[...] * (y - mu) * jax.lax.rsqrt(var + 1e-5) + bb_ref[...]


def _proj_body(x_ref, w_ref, b_ref, o_ref):
    o_ref[...] = (
        jnp.dot(x_ref[...], w_ref[...], preferred_element_type=jnp.float32)
        + b_ref[...]
    )


def _proj2_body(x_ref, w_ref, b_ref, a_ref, o_ref):
    y = jnp.dot(x_ref[...], w_ref[...], preferred_element_type=jnp.float32)
    y = y + b_ref[...]
    o_ref[...] = jnp.dot(y, a_ref[...], preferred_element_type=jnp.float32)


def _logit_body(kr_ref, q_ref, s_ref, p_ref, o_ref):
    prod = kr_ref[...] * q_ref[...]
    heads = jnp.dot(prod, s_ref[...], preferred_element_type=jnp.float32)
    o_ref[...] = heads * p_ref[...]


def _exp_body(l_ref, m_ref, o_ref):
    o_ref[...] = jnp.exp(l_ref[...] - m_ref[...])


def _msg_body(eu_ref, den_ref, vr_ref, r_ref, o_ref):
    den = den_ref[...]
    alpha = eu_ref[...] / jnp.where(den > 0, den, 1.0)
    a128 = jnp.dot(alpha, r_ref[...], preferred_element_type=jnp.float32)
    o_ref[...] = vr_ref[...] * a128


def _upd_body(agg_ref, h_ref, wa_ref, ba_ref, sk_ref, o_ref):
    o = jnp.dot(
        jax.nn.gelu(agg_ref[...]), wa_ref[...], preferred_element_type=jnp.float32
    )
    o = o + ba_ref[...]
    a = jax.nn.sigmoid(sk_ref[0, 0])
    new = a * o + (1.0 - a) * h_ref[...]
    o_ref[...] = jnp.maximum(new + h_ref[...], 0.0)


def _in_ln(x, w, b, g, bb):
    n = x.shape[0]
    return _call(
        _in_ln_body, n, _HID,
        (x, w, b.reshape(1, _HID), g.reshape(1, _HID), bb.reshape(1, _HID)),
        [_bs_rows(_HID), _bs_full(_HID, _HID), _bs_full(1, _HID),
         _bs_full(1, _HID), _bs_full(1, _HID)],
    )


def _proj(x, w, b):
    n = x.shape[0]
    return _call(
        _proj_body, n, _HID,
        (x, w, b.reshape(1, _HID)),
        [_bs_rows(_HID), _bs_full(_HID, _HID), _bs_full(1, _HID)],
    )


def _proj2(x, w, b, a):
    n = x.shape[0]
    return _call(
        _proj2_body, n, _HID,
        (x, w, b.reshape(1, _HID), a),
        [_bs_rows(_HID), _bs_full(_HID, _HID), _bs_full(1, _HID),
         _bs_full(_HID, _HID)],
    )


def _logits(kr_e, q_e, s_mat, p):
    return _call(
        _logit_body, _E, _H,
        (kr_e, q_e, s_mat, p.reshape(1, _H)),
        [_bs_rows(_HID), _bs_rows(_HID), _bs_full(_HID, _H), _bs_full(1, _H)],
    )


def _expm(logit, m_e):
    return _call(
        _exp_body, _E, _H,
        (logit, m_e),
        [_bs_rows(_H), _bs_rows(_H)],
    )


def _messages(eu, den_e, vr_e, r_mat):
    return _call(
        _msg_body, _E, _HID,
        (eu, den_e, vr_e, r_mat),
        [_bs_rows(_H), _bs_rows(_H), _bs_rows(_HID), _bs_full(_H, _HID)],
    )


def _update(agg, h, wa, ba, sk):
    n = h.shape[0]
    return _call(
        _upd_body, n, _HID,
        (agg, h, wa, ba.reshape(1, _HID), sk.reshape(1, 1)),
        [_bs_rows(_HID), _bs_rows(_HID), _bs_full(_HID, _HID),
         _bs_full(1, _HID), _bs_full(1, 1)],
    )


def _block_diag(a):
    eye = jnp.eye(_H, dtype=jnp.float32)
    return jnp.einsum('gh,hde->gdhe', eye, a).reshape(_H * _D, _H * _D)


def kernel(x_gene, x_trait, edge_g2t_src, edge_g2t_dst, edge_t2g_src,
           edge_t2g_dst, in_W, in_b, ln_g, ln_b, Wk, bk, Wq, bq, Wv, bv,
           Wa, ba, a_rel, m_rel, p_rel, skip, out_W, out_b):
    s_mat = jnp.repeat(jnp.eye(_H, dtype=jnp.float32), _D, axis=0)
    r_mat = jnp.repeat(jnp.eye(_H, dtype=jnp.float32), _D, axis=1)
    inv_sqrt_d = 1.0 / jnp.sqrt(jnp.float32(_D))

    hs = [
        _in_ln(x_gene, in_W[0], in_b[0], ln_g[0], ln_b[0]),
        _in_ln(x_trait, in_W[1], in_b[1], ln_g[1], ln_b[1]),
    ]
    edges = [(0, 1, edge_g2t_src, edge_g2t_dst),
             (1, 0, edge_t2g_src, edge_t2g_dst)]

    for l in range(_L):
        q = [_proj(hs[t], Wq[l, t], bq[l, t]) for t in (0, 1)]
        agg = [None, None]
        for e, (st, dt, si, di) in enumerate(edges):
            kr = _proj2(hs[st], Wk[l, st], bk[l, st], _block_diag(a_rel[l, e]))
            vr = _proj2(hs[st], Wv[l, st], bv[l, st], _block_diag(m_rel[l, e]))
            kr_e = jnp.take(kr, si, axis=0)
            q_e = jnp.take(q[dt], di, axis=0)
            vr_e = jnp.take(vr, si, axis=0)
            n_dst = hs[dt].shape[0]
            logit = _logits(kr_e, q_e, s_mat, p_rel[l, e] * inv_sqrt_d)
            m = jax.ops.segment_max(logit, di, num_segments=n_dst)
            m = jnp.where(jnp.isfinite(m), m, 0.0)
            eu = _expm(logit, jnp.take(m, di, axis=0))
            den = jax.ops.segment_sum(eu, di, num_segments=n_dst)
            msg = _messages(eu, jnp.take(den, di, axis=0), vr_e, r_mat)
            agg[dt] = jax.ops.segment_sum(msg, di, num_segments=n_dst)
        hs = [_update(agg[t], hs[t], Wa[l, t], ba[l, t], skip[l, t])
              for t in (0, 1)]

    return (_proj(hs[0], out_W[0], out_b[0]),
            _proj(hs[1], out_W[1], out_b[1]))


# speedup vs baseline: 7.1949x; 1.1230x over previous
import jax
import jax.numpy as jnp
from jax.experimental import pallas as pl

_NG, _NT, _E, _HID, _H, _D, _L = 50000, 5000, 300000, 128, 4, 32, 2
_BLK = 1024


def _bs_rows(cols):
    return pl.BlockSpec((_BLK, cols), lambda i: (i, 0))


def _bs_full(r, c):
    return pl.BlockSpec((r, c), lambda i: (0, 0))


def _call(body, n_rows, out_cols, operands, in_specs):
    return pl.pallas_call(
        body,
        grid=(pl.cdiv(n_rows, _BLK),),
        in_specs=in_specs,
        out_specs=_bs_rows(out_cols),
        out_shape=jax.ShapeDtypeStruct((n_rows, out_cols), jnp.float32),
    )(*operands)


def _in_ln_body(x_ref, w_ref, b_ref, g_ref, bb_ref, o_ref):
    y = jnp.dot(x_ref[...], w_ref[...], preferred_element_type=jnp.float32)
    y = y + b_ref[...]
    mu = y.mean(-1, keepdims=True)
    var = ((y - mu) ** 2).mean(-1, keepdims=True)
    o_ref[...] = g_ref[...] * (y - mu) * jax.lax.rsqrt(var + 1e-5) + bb_ref[...]


def _proj_body(x_ref, w_ref, b_ref, o_ref):
    o_ref[...] = (
        jnp.dot(x_ref[...], w_ref[...], preferred_element_type=jnp.float32)
        + b_ref[...]
    )


def _krv_body(x_ref, wk_ref, bk_ref, ak_ref, wv_ref, bv_ref, av_ref, o_ref):
    x = x_ref[...]
    k = jnp.dot(x, wk_ref[...], preferred_element_type=jnp.float32) + bk_ref[...]
    v = jnp.dot(x, wv_ref[...], preferred_element_type=jnp.float32) + bv_ref[...]
    o_ref[:, :_HID] = jnp.dot(k, ak_ref[...], preferred_element_type=jnp.float32)
    o_ref[:, _HID:] = jnp.dot(v, av_ref[...], preferred_element_type=jnp.float32)


def _logit_body(krv_ref, q_ref, s_ref, p_ref, o_ref):
    prod = krv_ref[:, :_HID] * q_ref[...]
    heads = jnp.dot(prod, s_ref[...], preferred_element_type=jnp.float32)
    o_ref[...] = heads * p_ref[...]


def _exp_body(l_ref, m_ref, o_ref):
    o_ref[...] = jnp.exp(l_ref[...] - m_ref[...])


def _msg_body(eu_ref, den_ref, krv_ref, r_ref, o_ref):
    den = den_ref[...]
    alpha = eu_ref[...] / jnp.where(den > 0, den, 1.0)
    a128 = jnp.dot(alpha, r_ref[...], preferred_element_type=jnp.float32)
    o_ref[...] = krv_ref[:, _HID:] * a128


def _upd_body(agg_ref, h_ref, wa_ref, ba_ref, sk_ref, o_ref):
    o = jnp.dot(
        jax.nn.gelu(agg_ref[...]), wa_ref[...], preferred_element_type=jnp.float32
    )
    o = o + ba_ref[...]
    a = jax.nn.sigmoid(sk_ref[0, 0])
    new = a * o + (1.0 - a) * h_ref[...]
    o_ref[...] = jnp.maximum(new + h_ref[...], 0.0)


def _in_ln(x, w, b, g, bb):
    n = x.shape[0]
    return _call(
        _in_ln_body, n, _HID,
        (x, w, b.reshape(1, _HID), g.reshape(1, _HID), bb.reshape(1, _HID)),
        [_bs_rows(_HID), _bs_full(_HID, _HID), _bs_full(1, _HID),
         _bs_full(1, _HID), _bs_full(1, _HID)],
    )


def _proj(x, w, b):
    n = x.shape[0]
    return _call(
        _proj_body, n, _HID,
        (x, w, b.reshape(1, _HID)),
        [_bs_rows(_HID), _bs_full(_HID, _HID), _bs_full(1, _HID)],
    )


def _krv(x, wk, bk, ak, wv, bv, av):
    n = x.shape[0]
    return _call(
        _krv_body, n, 2 * _HID,
        (x, wk, bk.reshape(1, _HID), ak, wv, bv.reshape(1, _HID), av),
        [_bs_rows(_HID), _bs_full(_HID, _HID), _bs_full(1, _HID),
         _bs_full(_HID, _HID), _bs_full(_HID, _HID), _bs_full(1, _HID),
         _bs_full(_HID, _HID)],
    )


def _logits(krv_e, q_e, s_mat, p):
    return _call(
        _logit_body, _E, _H,
        (krv_e, q_e, s_mat, p.reshape(1, _H)),
        [_bs_rows(2 * _HID), _bs_rows(_HID), _bs_full(_HID, _H),
         _bs_full(1, _H)],
    )


def _expm(logit, m_e):
    return _call(
        _exp_body, _E, _H,
        (logit, m_e),
        [_bs_rows(_H), _bs_rows(_H)],
    )


def _messages(eu, den_e, krv_e, r_mat):
    return _call(
        _msg_body, _E, _HID,
        (eu, den_e, krv_e, r_mat),
        [_bs_rows(_H), _bs_rows(_H), _bs_rows(2 * _HID), _bs_full(_H, _HID)],
    )


def _update(agg, h, wa, ba, sk):
    n = h.shape[0]
    return _call(
        _upd_body, n, _HID,
        (agg, h, wa, ba.reshape(1, _HID), sk.reshape(1, 1)),
        [_bs_rows(_HID), _bs_rows(_HID), _bs_full(_HID, _HID),
         _bs_full(1, _HID), _bs_full(1, 1)],
    )


def _block_diag(a):
    eye = jnp.eye(_H, dtype=jnp.float32)
    return jnp.einsum('gh,hde->gdhe', eye, a).reshape(_H * _D, _H * _D)


def kernel(x_gene, x_trait, edge_g2t_src, edge_g2t_dst, edge_t2g_src,
           edge_t2g_dst, in_W, in_b, ln_g, ln_b, Wk, bk, Wq, bq, Wv, bv,
           Wa, ba, a_rel, m_rel, p_rel, skip, out_W, out_b):
    s_mat = jnp.repeat(jnp.eye(_H, dtype=jnp.float32), _D, axis=0)
    r_mat = jnp.repeat(jnp.eye(_H, dtype=jnp.float32), _D, axis=1)
    inv_sqrt_d = 1.0 / jnp.sqrt(jnp.float32(_D))

    hs = [
        _in_ln(x_gene, in_W[0], in_b[0], ln_g[0], ln_b[0]),
        _in_ln(x_trait, in_W[1], in_b[1], ln_g[1], ln_b[1]),
    ]
    edges = [(0, 1, edge_g2t_src, edge_g2t_dst),
             (1, 0, edge_t2g_src, edge_t2g_dst)]

    for l in range(_L):
        q = [_proj(hs[t], Wq[l, t], bq[l, t]) for t in (0, 1)]
        agg = [None, None]
        for e, (st, dt, si, di) in enumerate(edges):
            krv = _krv(hs[st], Wk[l, st], bk[l, st], _block_diag(a_rel[l, e]),
                       Wv[l, st], bv[l, st], _block_diag(m_rel[l, e]))
            krv_e = jnp.take(krv, si, axis=0)
            q_e = jnp.take(q[dt], di, axis=0)
            n_dst = hs[dt].shape[0]
            logit = _logits(krv_e, q_e, s_mat, p_rel[l, e] * inv_sqrt_d)
            m = jax.ops.segment_max(logit, di, num_segments=n_dst)
            m = jnp.where(jnp.isfinite(m), m, 0.0)
            eu = _expm(logit, jnp.take(m, di, axis=0))
            den = jax.ops.segment_sum(eu, di, num_segments=n_dst)
            msg = _messages(eu, jnp.take(den, di, axis=0), krv_e, r_mat)
            agg[dt] = jax.ops.segment_sum(msg, di, num_segments=n_dst)
        hs = [_update(agg[t], hs[t], Wa[l, t], ba[l, t], skip[l, t])
              for t in (0, 1)]

    return (_proj(hs[0], out_W[0], out_b[0]),
            _proj(hs[1], out_W[1], out_b[1]))


# speedup vs baseline: 7.4003x; 1.0286x over previous
import jax
import jax.numpy as jnp
from jax.experimental import pallas as pl

_NG, _NT, _E, _HID, _H, _D, _L = 50000, 5000, 300000, 128, 4, 32, 2
_BLK = 2048


def _bs_rows(cols):
    return pl.BlockSpec((_BLK, cols), lambda i: (i, 0))


def _bs_full(r, c):
    return pl.BlockSpec((r, c), lambda i: (0, 0))


def _call(body, n_rows, out_cols, operands, in_specs):
    return pl.pallas_call(
        body,
        grid=(pl.cdiv(n_rows, _BLK),),
        in_specs=in_specs,
        out_specs=_bs_rows(out_cols),
        out_shape=jax.ShapeDtypeStruct((n_rows, out_cols), jnp.float32),
    )(*operands)


def _in_ln_body(x_ref, w_ref, b_ref, g_ref, bb_ref, o_ref):
    y = jnp.dot(x_ref[...], w_ref[...], preferred_element_type=jnp.float32)
    y = y + b_ref[...]
    mu = y.mean(-1, keepdims=True)
    var = ((y - mu) ** 2).mean(-1, keepdims=True)
    o_ref[...] = g_ref[...] * (y - mu) * jax.lax.rsqrt(var + 1e-5) + bb_ref[...]


def _proj_body(x_ref, w_ref, b_ref, o_ref):
    o_ref[...] = (
        jnp.dot(x_ref[...], w_ref[...], preferred_element_type=jnp.float32)
        + b_ref[...]
    )


def _krv_body(x_ref, wk_ref, bk_ref, ak_ref, wv_ref, bv_ref, av_ref, o_ref):
    x = x_ref[...]
    k = jnp.dot(x, wk_ref[...], preferred_element_type=jnp.float32) + bk_ref[...]
    v = jnp.dot(x, wv_ref[...], preferred_element_type=jnp.float32) + bv_ref[...]
    o_ref[:, :_HID] = jnp.dot(k, ak_ref[...], preferred_element_type=jnp.float32)
    o_ref[:, _HID:] = jnp.dot(v, av_ref[...], preferred_element_type=jnp.float32)


def _logit_body(krv_ref, q_ref, s_ref, p_ref, o_ref):
    prod = krv_ref[:, :_HID] * q_ref[...]
    heads = jnp.dot(prod, s_ref[...], preferred_element_type=jnp.float32)
    o_ref[...] = heads * p_ref[...]


def _exp_body(l_ref, m_ref, o_ref):
    o_ref[...] = jnp.exp(l_ref[...] - m_ref[...])


def _msg_body(eu_ref, den_ref, krv_ref, r_ref, o_ref):
    den = den_ref[...]
    alpha = eu_ref[...] / jnp.where(den > 0, den, 1.0)
    a128 = jnp.dot(alpha, r_ref[...], preferred_element_type=jnp.float32)
    o_ref[...] = krv_ref[:, _HID:] * a128


def _upd_body(agg_ref, h_ref, wa_ref, ba_ref, sk_ref, o_ref):
    o = jnp.dot(
        jax.nn.gelu(agg_ref[...]), wa_ref[...], preferred_element_type=jnp.float32
    )
    o = o + ba_ref[...]
    a = jax.nn.sigmoid(sk_ref[0, 0])
    new = a * o + (1.0 - a) * h_ref[...]
    o_ref[...] = jnp.maximum(new + h_ref[...], 0.0)


def _in_ln(x, w, b, g, bb):
    n = x.shape[0]
    return _call(
        _in_ln_body, n, _HID,
        (x, w, b.reshape(1, _HID), g.reshape(1, _HID), bb.reshape(1, _HID)),
        [_bs_rows(_HID), _bs_full(_HID, _HID), _bs_full(1, _HID),
         _bs_full(1, _HID), _bs_full(1, _HID)],
    )


def _proj(x, w, b):
    n = x.shape[0]
    return _call(
        _proj_body, n, _HID,
        (x, w, b.reshape(1, _HID)),
        [_bs_rows(_HID), _bs_full(_HID, _HID), _bs_full(1, _HID)],
    )


def _krv(x, wk, bk, ak, wv, bv, av):
    n = x.shape[0]
    return _call(
        _krv_body, n, 2 * _HID,
        (x, wk, bk.reshape(1, _HID), ak, wv, bv.reshape(1, _HID), av),
        [_bs_rows(_HID), _bs_full(_HID, _HID), _bs_full(1, _HID),
         _bs_full(_HID, _HID), _bs_full(_HID, _HID), _bs_full(1, _HID),
         _bs_full(_HID, _HID)],
    )


def _logits(krv_e, q_e, s_mat, p):
    return _call(
        _logit_body, _E, _H,
        (krv_e, q_e, s_mat, p.reshape(1, _H)),
        [_bs_rows(2 * _HID), _bs_rows(_HID), _bs_full(_HID, _H),
         _bs_full(1, _H)],
    )


def _expm(logit, m_e):
    return _call(
        _exp_body, _E, _H,
        (logit, m_e),
        [_bs_rows(_H), _bs_rows(_H)],
    )


def _messages(eu, den_e, krv_e, r_mat):
    return _call(
        _msg_body, _E, _HID,
        (eu, den_e, krv_e, r_mat),
        [_bs_rows(_H), _bs_rows(_H), _bs_rows(2 * _HID), _bs_full(_H, _HID)],
    )


def _update(agg, h, wa, ba, sk):
    n = h.shape[0]
    return _call(
        _upd_body, n, _HID,
        (agg, h, wa, ba.reshape(1, _HID), sk.reshape(1, 1)),
        [_bs_rows(_HID), _bs_rows(_HID), _bs_full(_HID, _HID),
         _bs_full(1, _HID), _bs_full(1, 1)],
    )


def _block_diag(a):
    eye = jnp.eye(_H, dtype=jnp.float32)
    return jnp.einsum('gh,hde->gdhe', eye, a).reshape(_H * _D, _H * _D)


def kernel(x_gene, x_trait, edge_g2t_src, edge_g2t_dst, edge_t2g_src,
           edge_t2g_dst, in_W, in_b, ln_g, ln_b, Wk, bk, Wq, bq, Wv, bv,
           Wa, ba, a_rel, m_rel, p_rel, skip, out_W, out_b):
    s_mat = jnp.repeat(jnp.eye(_H, dtype=jnp.float32), _D, axis=0)
    r_mat = jnp.repeat(jnp.eye(_H, dtype=jnp.float32), _D, axis=1)
    inv_sqrt_d = 1.0 / jnp.sqrt(jnp.float32(_D))

    hs = [
        _in_ln(x_gene, in_W[0], in_b[0], ln_g[0], ln_b[0]),
        _in_ln(x_trait, in_W[1], in_b[1], ln_g[1], ln_b[1]),
    ]
    edges = [(0, 1, edge_g2t_src, edge_g2t_dst),
             (1, 0, edge_t2g_src, edge_t2g_dst)]

    for l in range(_L):
        q = [_proj(hs[t], Wq[l, t], bq[l, t]) for t in (0, 1)]
        agg = [None, None]
        for e, (st, dt, si, di) in enumerate(edges):
            krv = _krv(hs[st], Wk[l, st], bk[l, st], _block_diag(a_rel[l, e]),
                       Wv[l, st], bv[l, st], _block_diag(m_rel[l, e]))
            krv_e = jnp.take(krv, si, axis=0)
            q_e = jnp.take(q[dt], di, axis=0)
            n_dst = hs[dt].shape[0]
            logit = _logits(krv_e, q_e, s_mat, p_rel[l, e] * inv_sqrt_d)
            m = jax.ops.segment_max(logit, di, num_segments=n_dst)
            m = jnp.where(jnp.isfinite(m), m, 0.0)
            eu = _expm(logit, jnp.take(m, di, axis=0))
            den = jax.ops.segment_sum(eu, di, num_segments=n_dst)
            msg = _messages(eu, jnp.take(den, di, axis=0), krv_e, r_mat)
            agg[dt] = jax.ops.segment_sum(msg, di, num_segments=n_dst)
        hs = [_update(agg[t], hs[t], Wa[l, t], ba[l, t], skip[l, t])
              for t in (0, 1)]

    return (_proj(hs[0], out_W[0], out_b[0]),
            _proj(hs[1], out_W[1], out_b[1]))
